# baseline (speedup 1.0000x reference)
import functools

import jax
import jax.numpy as jnp
from jax import lax
from jax.experimental import pallas as pl
from jax.experimental.pallas import tpu as pltpu
from jax.experimental.pallas import tpu_sc as plsc

N_NODES = 10000
D_FEAT = 128
N_GRAPHS = 64
CHUNK = 320
TAIL = N_NODES - 31 * CHUNK
NW = 32


def _seg_body(x_hbm, b_hbm, out_sc, xv, bv, sp, cp, sem0, sem1):
    cid = lax.axis_index("c")
    sid = lax.axis_index("s")
    wid = cid * 16 + sid
    iota = lax.iota(jnp.int32, 16)
    zero16 = jnp.zeros((16,), jnp.float32)
    ones16 = jnp.ones((16,), jnp.float32)

    for l in range(16):
        for gg in range(N_GRAPHS // 16):
            sp[l, pl.ds(gg * 16, 16)] = zero16
            cp[l, pl.ds(gg * 16, 16)] = zero16

    ms = [iota ^ k for k in range(16)]

    def group_body(t, _):
        fb = t * (16 * D_FEAT) + iota * D_FEAT
        bvec = bv[pl.ds(pl.multiple_of(t * 16, 16), 16)]
        msg = [fb + m for m in ms]

        def dstep(j, carry):
            a0, a1, a2, a3 = carry
            dsp = jnp.full((16,), j * 16, jnp.int32)
            accs = [a0, a1, a2, a3]
            for k in range(16):
                g = plsc.load_gather(xv, [msg[k] + dsp])
                accs[k % 4] = accs[k % 4] + g
            return tuple(accs)

        accs = lax.fori_loop(0, D_FEAT // 16, dstep,
                             (zero16, zero16, zero16, zero16))
        acc = (accs[0] + accs[1]) + (accs[2] + accs[3])
        plsc.addupdate_scatter(sp, [iota, bvec], acc)
        plsc.addupdate_scatter(cp, [iota, bvec], ones16)
        return 0

    QROWS = CHUNK // 4

    @pl.when(wid < NW - 1)
    def _():
        base = wid * CHUNK
        copies = []
        for q in range(4):
            copies.append(pltpu.async_copy(
                x_hbm.at[pl.ds((base + q * QROWS) * D_FEAT, QROWS * D_FEAT)],
                xv.at[pl.ds(q * QROWS * D_FEAT, QROWS * D_FEAT)],
                sem0 if q % 2 == 0 else sem1))
        pltpu.sync_copy(b_hbm.at[pl.ds(base, CHUNK)], bv)
        for q in range(4):
            copies[q].wait()
            lax.fori_loop(q * (QROWS // 16), (q + 1) * (QROWS // 16),
                          group_body, 0)

    @pl.when(wid == NW - 1)
    def _():
        base = (NW - 1) * CHUNK
        pltpu.sync_copy(x_hbm.at[pl.ds(base * D_FEAT, TAIL * D_FEAT)],
                        xv.at[pl.ds(0, TAIL * D_FEAT)])
        pltpu.sync_copy(b_hbm.at[pl.ds(base, TAIL)], bv.at[pl.ds(0, TAIL)])
        lax.fori_loop(0, TAIL // 16, group_body, 0)

    for gg in range(N_GRAPHS // 16):
        acc_s = sp[0, pl.ds(gg * 16, 16)]
        acc_c = cp[0, pl.ds(gg * 16, 16)]
        for l in range(1, 16):
            acc_s = acc_s + sp[l, pl.ds(gg * 16, 16)]
            acc_c = acc_c + cp[l, pl.ds(gg * 16, 16)]
        sp[0, pl.ds(gg * 16, 16)] = acc_s
        cp[0, pl.ds(gg * 16, 16)] = acc_c
    pltpu.sync_copy(sp.at[0], out_sc.at[wid])
    pltpu.sync_copy(cp.at[0], out_sc.at[NW + wid])


_seg = functools.partial(
    pl.kernel,
    out_type=jax.ShapeDtypeStruct((2 * NW, N_GRAPHS), jnp.float32),
    mesh=plsc.VectorSubcoreMesh(core_axis_name="c", subcore_axis_name="s"),
    compiler_params=pltpu.CompilerParams(needs_layout_passes=False),
    scratch_types=[
        pltpu.VMEM((CHUNK * D_FEAT,), jnp.float32),
        pltpu.VMEM((CHUNK,), jnp.int32),
        pltpu.VMEM((16, N_GRAPHS), jnp.float32),
        pltpu.VMEM((16, N_GRAPHS), jnp.float32),
        pltpu.SemaphoreType.DMA,
        pltpu.SemaphoreType.DMA,
    ],
)(_seg_body)


def _combine_body(sc_ref, u_ref, o_ref):
    s = jnp.sum(sc_ref[0:NW, :], axis=0, keepdims=True)
    c = jnp.sum(sc_ref[NW:2 * NW, :], axis=0, keepdims=True)
    ones_row = jnp.ones((1, u_ref.shape[1]), jnp.float32)
    us = lax.dot_general(ones_row, u_ref[...],
                         (((1,), (1,)), ((), ())))
    denom = jnp.float32(u_ref.shape[1] + D_FEAT)
    o_ref[...] = (us + s / jnp.maximum(c, 1.0)) / denom


def kernel(x, edge_index, edge_attr, u, batch):
    del edge_index, edge_attr
    b = batch.astype(jnp.int32)
    part_sc = _seg(x.reshape(-1), b)
    out = pl.pallas_call(
        _combine_body,
        out_shape=jax.ShapeDtypeStruct((1, N_GRAPHS), jnp.float32),
    )(part_sc, u)
    return out.reshape(N_GRAPHS)


# speedup vs baseline: 1.0024x; 1.0024x over previous
import functools

import jax
import jax.numpy as jnp
from jax import lax
from jax.experimental import pallas as pl
from jax.experimental.pallas import tpu as pltpu
from jax.experimental.pallas import tpu_sc as plsc

N_NODES = 10000
D_FEAT = 128
N_GRAPHS = 64
CHUNK = 320
TAIL = N_NODES - 31 * CHUNK
NW = 32


def _seg_body(x_hbm, b_hbm, out_sc, xv, bv, sp, cp, sem0, sem1):
    cid = lax.axis_index("c")
    sid = lax.axis_index("s")
    wid = cid * 16 + sid
    iota = lax.iota(jnp.int32, 16)
    zero16 = jnp.zeros((16,), jnp.float32)
    ones16 = jnp.ones((16,), jnp.float32)

    for l in range(16):
        for gg in range(N_GRAPHS // 16):
            sp[l, pl.ds(gg * 16, 16)] = zero16
            cp[l, pl.ds(gg * 16, 16)] = zero16

    ms = [iota ^ k for k in range(16)]

    def group_body(t, _):
        fb = t * (16 * D_FEAT) + iota * D_FEAT
        bvec = bv[pl.ds(pl.multiple_of(t * 16, 16), 16)]
        msg = [fb + m for m in ms]

        def dstep(j, carry):
            a0, a1, a2, a3 = carry
            dsp = jnp.full((16,), j * 16, jnp.int32)
            accs = [a0, a1, a2, a3]
            for k in range(16):
                g = plsc.load_gather(xv, [msg[k] + dsp])
                accs[k % 4] = accs[k % 4] + g
            return tuple(accs)

        accs = plsc.parallel_loop(
            0, D_FEAT // 16, carry=(zero16, zero16, zero16, zero16))(dstep)
        acc = (accs[0] + accs[1]) + (accs[2] + accs[3])
        plsc.addupdate_scatter(sp, [iota, bvec], acc)
        plsc.addupdate_scatter(cp, [iota, bvec], ones16)
        return 0

    QROWS = CHUNK // 4

    @pl.when(wid < NW - 1)
    def _():
        base = wid * CHUNK
        copies = []
        for q in range(4):
            copies.append(pltpu.async_copy(
                x_hbm.at[pl.ds((base + q * QROWS) * D_FEAT, QROWS * D_FEAT)],
                xv.at[pl.ds(q * QROWS * D_FEAT, QROWS * D_FEAT)],
                sem0 if q % 2 == 0 else sem1))
        pltpu.sync_copy(b_hbm.at[pl.ds(base, CHUNK)], bv)
        for q in range(4):
            copies[q].wait()
            lax.fori_loop(q * (QROWS // 16), (q + 1) * (QROWS // 16),
                          group_body, 0)

    @pl.when(wid == NW - 1)
    def _():
        base = (NW - 1) * CHUNK
        pltpu.sync_copy(x_hbm.at[pl.ds(base * D_FEAT, TAIL * D_FEAT)],
                        xv.at[pl.ds(0, TAIL * D_FEAT)])
        pltpu.sync_copy(b_hbm.at[pl.ds(base, TAIL)], bv.at[pl.ds(0, TAIL)])
        lax.fori_loop(0, TAIL // 16, group_body, 0)

    for gg in range(N_GRAPHS // 16):
        acc_s = sp[0, pl.ds(gg * 16, 16)]
        acc_c = cp[0, pl.ds(gg * 16, 16)]
        for l in range(1, 16):
            acc_s = acc_s + sp[l, pl.ds(gg * 16, 16)]
            acc_c = acc_c + cp[l, pl.ds(gg * 16, 16)]
        sp[0, pl.ds(gg * 16, 16)] = acc_s
        cp[0, pl.ds(gg * 16, 16)] = acc_c
    pltpu.sync_copy(sp.at[0], out_sc.at[wid])
    pltpu.sync_copy(cp.at[0], out_sc.at[NW + wid])


_seg = functools.partial(
    pl.kernel,
    out_type=jax.ShapeDtypeStruct((2 * NW, N_GRAPHS), jnp.float32),
    mesh=plsc.VectorSubcoreMesh(core_axis_name="c", subcore_axis_name="s"),
    compiler_params=pltpu.CompilerParams(needs_layout_passes=False),
    scratch_types=[
        pltpu.VMEM((CHUNK * D_FEAT,), jnp.float32),
        pltpu.VMEM((CHUNK,), jnp.int32),
        pltpu.VMEM((16, N_GRAPHS), jnp.float32),
        pltpu.VMEM((16, N_GRAPHS), jnp.float32),
        pltpu.SemaphoreType.DMA,
        pltpu.SemaphoreType.DMA,
    ],
)(_seg_body)


def _combine_body(sc_ref, u_ref, o_ref):
    s = jnp.sum(sc_ref[0:NW, :], axis=0, keepdims=True)
    c = jnp.sum(sc_ref[NW:2 * NW, :], axis=0, keepdims=True)
    ones_row = jnp.ones((1, u_ref.shape[1]), jnp.float32)
    us = lax.dot_general(ones_row, u_ref[...],
                         (((1,), (1,)), ((), ())))
    denom = jnp.float32(u_ref.shape[1] + D_FEAT)
    o_ref[...] = (us + s / jnp.maximum(c, 1.0)) / denom


def kernel(x, edge_index, edge_attr, u, batch):
    del edge_index, edge_attr
    b = batch.astype(jnp.int32)
    part_sc = _seg(x.reshape(-1), b)
    out = pl.pallas_call(
        _combine_body,
        out_shape=jax.ShapeDtypeStruct((1, N_GRAPHS), jnp.float32),
    )(part_sc, u)
    return out.reshape(N_GRAPHS)
